# baseline (speedup 1.0000x reference)
import jax
import jax.numpy as jnp
from jax import lax
from jax.experimental import pallas as pl
from jax.experimental.pallas import tpu as pltpu
from jax.experimental.pallas import tpu_sc as plsc

NC = 2
NS = 16
L = 16
NW = NC * NS

B = 16384
D = 64

B_SC = 8192
B_TC = B - B_SC

B_PER_W = B_SC // NW
CH = 128
NCHUNK = B_PER_W // CH
GPC = CH // L

BLK = 256
NBLK = B_TC // BLK


def _sc_body(i_hbm, j_hbm, h_hbm, c_hbm, out_hbm,
             idx_i, idx_j, hb0, cb0, hb1, cb1, out_v, sem0, sem1):
    wid = lax.axis_index("s") * NC + lax.axis_index("c")
    base = wid * B_PER_W

    pltpu.sync_copy(i_hbm.at[pl.ds(base, B_PER_W)], idx_i)
    pltpu.sync_copy(j_hbm.at[pl.ds(base, B_PER_W)], idx_j)

    lane = lax.iota(jnp.int32, L)
    hbufs = (hb0, hb1)
    cbufs = (cb0, cb1)
    sems = (sem0, sem1)

    def fetch_chunk(ch, hb, cb, sem):
        def fetch(g, _c):
            iv = idx_i[pl.ds(ch * CH + g * L, L)]
            jv = idx_j[pl.ds(ch * CH + g * L, L)]
            for t in range(L):
                pltpu.make_async_copy(
                    h_hbm.at[pl.ds(iv[t], 1), :],
                    hb.at[pl.ds(g * L + t, 1), :], sem).start()
                pltpu.make_async_copy(
                    c_hbm.at[pl.ds(jv[t], 1), :],
                    cb.at[pl.ds(g * L + t, 1), :], sem).start()
            return ()

        lax.fori_loop(0, GPC, fetch, (), unroll=False)

    def drain_chunk(hb, cb, sem):
        pltpu.make_async_copy(h_hbm.at[pl.ds(0, CH), :], hb, sem).wait()
        pltpu.make_async_copy(c_hbm.at[pl.ds(0, CH), :], cb, sem).wait()

    def compute_chunk(ch, hb, cb):
        for lg in range(GPC):
            rows = lg * L + lane
            acc = jnp.zeros((L,), jnp.float32)
            dvec = jnp.zeros((L,), jnp.int32)
            for _step in range(D):
                hv = plsc.load_gather(hb, [rows, dvec])
                cv = plsc.load_gather(cb, [rows, dvec])
                acc = acc + hv * cv
                dvec = dvec + 1
            sig = 1.0 / (1.0 + jnp.exp(-acc))
            out_v[pl.ds(ch * CH + lg * L, L)] = sig

    fetch_chunk(0, hb0, cb0, sem0)
    for ch in range(NCHUNK):
        pb = ch % 2
        if ch + 1 < NCHUNK:
            fetch_chunk(ch + 1, hbufs[1 - pb], cbufs[1 - pb], sems[1 - pb])
        drain_chunk(hbufs[pb], cbufs[pb], sems[pb])
        compute_chunk(ch, hbufs[pb], cbufs[pb])

    pltpu.sync_copy(out_v, out_hbm.at[pl.ds(base, B_PER_W)])


def _tc_body(idx_i_smem, idx_j_smem, h_hbm, c_hbm, out_ref,
             hb0, cb0, hb1, cb1, sem0, sem1):

    def fetch_block(blk, hb, cb, sem):
        def fetch(r, _):
            ri = idx_i_smem[blk * BLK + r]
            rj = idx_j_smem[blk * BLK + r]
            pltpu.make_async_copy(
                h_hbm.at[pl.ds(ri, 1), :], hb.at[pl.ds(r, 1), :], sem).start()
            pltpu.make_async_copy(
                c_hbm.at[pl.ds(rj, 1), :], cb.at[pl.ds(r, 1), :], sem).start()
            return ()
        lax.fori_loop(0, BLK, fetch, (), unroll=8)

    def drain_block(hb, cb, sem):
        pltpu.make_async_copy(h_hbm.at[pl.ds(0, BLK), :], hb, sem).wait()
        pltpu.make_async_copy(c_hbm.at[pl.ds(0, BLK), :], cb, sem).wait()

    def compute_block(blk, hb, cb):
        prod = hb[...] * cb[...]
        s = jnp.sum(prod, axis=1)
        out_ref[pl.ds(blk * BLK, BLK)] = jax.nn.sigmoid(s)

    fetch_block(0, hb0, cb0, sem0)

    def step(k, _):
        blk0 = k * 2
        blk1 = k * 2 + 1
        fetch_block(blk1, hb1, cb1, sem1)
        drain_block(hb0, cb0, sem0)
        compute_block(blk0, hb0, cb0)

        @pl.when(blk1 + 1 < NBLK)
        def _():
            fetch_block(blk1 + 1, hb0, cb0, sem0)

        drain_block(hb1, cb1, sem1)
        compute_block(blk1, hb1, cb1)
        return ()

    lax.fori_loop(0, NBLK // 2, step, (), unroll=False)


@jax.jit
def kernel(i, j, H, C):
    i = i.astype(jnp.int32)
    j = j.astype(jnp.int32)

    mesh = plsc.VectorSubcoreMesh(
        core_axis_name="c", subcore_axis_name="s",
        num_cores=NC, num_subcores=NS)
    sc_run = pl.kernel(
        _sc_body,
        out_type=jax.ShapeDtypeStruct((B_SC,), jnp.float32),
        mesh=mesh,
        scratch_types=[
            pltpu.VMEM((B_PER_W,), jnp.int32),
            pltpu.VMEM((B_PER_W,), jnp.int32),
            pltpu.VMEM((CH, D), jnp.float32),
            pltpu.VMEM((CH, D), jnp.float32),
            pltpu.VMEM((CH, D), jnp.float32),
            pltpu.VMEM((CH, D), jnp.float32),
            pltpu.VMEM((B_PER_W,), jnp.float32),
            pltpu.SemaphoreType.DMA,
            pltpu.SemaphoreType.DMA,
        ],
        compiler_params=pltpu.CompilerParams(
            needs_layout_passes=False),
    )
    out_sc = sc_run(i[:B_SC], j[:B_SC], H, C)

    grid_spec = pltpu.PrefetchScalarGridSpec(
        num_scalar_prefetch=2,
        grid=(1,),
        in_specs=[
            pl.BlockSpec(memory_space=pltpu.HBM),
            pl.BlockSpec(memory_space=pltpu.HBM),
        ],
        out_specs=pl.BlockSpec(memory_space=pltpu.VMEM),
        scratch_shapes=[
            pltpu.VMEM((BLK, D), jnp.float32),
            pltpu.VMEM((BLK, D), jnp.float32),
            pltpu.VMEM((BLK, D), jnp.float32),
            pltpu.VMEM((BLK, D), jnp.float32),
            pltpu.SemaphoreType.DMA,
            pltpu.SemaphoreType.DMA,
        ],
    )
    tc_run = pl.pallas_call(
        _tc_body,
        grid_spec=grid_spec,
        out_shape=jax.ShapeDtypeStruct((B_TC,), jnp.float32),
    )
    out_tc = tc_run(i[B_SC:], j[B_SC:], H, C)

    return jnp.concatenate([out_sc, out_tc])
